# baseline (speedup 1.0000x reference)
import jax
import jax.numpy as jnp
from jax import lax
from jax.experimental import pallas as pl
from jax.experimental.pallas import tpu as pltpu
from jax.experimental.pallas import tpu_sc as plsc

NUM_CLASSES = 100000
HIDDEN_DIM = 128
BATCH = 16384

_INFO = plsc.get_sparse_core_info()
_NC = _INFO.num_cores
_NS = _INFO.num_subcores
_L = _INFO.num_lanes
_NW = _NC * _NS
_BPW = BATCH // _NW
_K = 64
_NCH = _BPW // _K
_GPC = _K // _L
_NCOL = HIDDEN_DIM // _L


def _sc_kernel(labels_hbm, drops_hbm, table_hbm, out_hbm, idx_v, drop_v,
               rows_v, row0_v, *sems):
    gsem, osem = sems[:_NCH], sems[_NCH]
    wid = lax.axis_index("s") * _NC + lax.axis_index("c")
    base = wid * _BPW

    pltpu.sync_copy(labels_hbm.at[pl.ds(wid * _NCH, _NCH)], idx_v)
    pltpu.sync_copy(drops_hbm.at[pl.ds(wid * _NCH, _NCH)], drop_v)
    row0_copy = pltpu.async_copy(table_hbm.at[pl.ds(0, 1)], row0_v, osem)

    lane = lax.iota(jnp.int32, _L)
    gathers = []
    for j in range(_NCH):
        for g in range(_GPC):
            sl = pl.ds(g * _L, _L)
            pos = lane + j * _K + g * _L
            idx_v[j, sl] = jnp.where(drop_v[j, sl] == 1, pos + base,
                                     idx_v[j, sl])
        gathers.append(pltpu.async_copy(
            table_hbm.at[idx_v.at[j]], rows_v.at[pl.ds(j * _K, _K)],
            gsem[j]))

    row0_copy.wait()
    r0 = [row0_v[0, pl.ds(c * _L, _L)] for c in range(_NCOL)]

    out_copies = []
    for j in range(_NCH):
        gathers[j].wait()

        def _fill(g, carry, j=j):
            drop16 = drop_v[j, pl.ds(g * _L, _L)]
            pos = lane + j * _K + g * _L
            tgt = jnp.where(drop16 == 1, pos, _BPW)
            for k in range(_L):
                r = tgt[k]
                for c in range(_NCOL):
                    rows_v[r, pl.ds(c * _L, _L)] = carry[c]
            return carry

        lax.fori_loop(0, _GPC, _fill, tuple(r0))
        out_copies.append(pltpu.async_copy(
            rows_v.at[pl.ds(j * _K, _K)],
            out_hbm.at[pl.ds(base + j * _K, _K)], osem))
    for c in out_copies:
        c.wait()


@jax.jit
def _embed(labels, force_drop_ids, embedding_table):
    mesh = plsc.VectorSubcoreMesh(core_axis_name="c", subcore_axis_name="s")
    return pl.kernel(
        _sc_kernel,
        mesh=mesh,
        out_type=jax.ShapeDtypeStruct((BATCH, HIDDEN_DIM), jnp.float32),
        scratch_types=[
            pltpu.VMEM((_NCH, _K), jnp.int32),
            pltpu.VMEM((_NCH, _K), jnp.int32),
            pltpu.VMEM((_BPW + 1, HIDDEN_DIM), jnp.float32),
            pltpu.VMEM((1, HIDDEN_DIM), jnp.float32),
        ] + [pltpu.SemaphoreType.DMA] * (_NCH + 1),
    )(labels.reshape(BATCH // _K, _K), force_drop_ids.reshape(BATCH // _K, _K),
      embedding_table)


def kernel(labels, train, force_drop_ids, embedding_table):
    del train
    return _embed(labels.astype(jnp.int32), force_drop_ids.astype(jnp.int32),
                  embedding_table)


# speedup vs baseline: 1.0401x; 1.0401x over previous
import jax
import jax.numpy as jnp
from jax import lax
from jax.experimental import pallas as pl
from jax.experimental.pallas import tpu as pltpu
from jax.experimental.pallas import tpu_sc as plsc

NUM_CLASSES = 100000
HIDDEN_DIM = 128
BATCH = 16384

_INFO = plsc.get_sparse_core_info()
_NC = _INFO.num_cores
_NS = _INFO.num_subcores
_L = _INFO.num_lanes
_NW = _NC * _NS
_BPW = BATCH // _NW
_K = 128
_NCH = _BPW // _K
_GPC = _K // _L
_NCOL = HIDDEN_DIM // _L


def _sc_kernel(labels_hbm, drops_hbm, table_hbm, out_hbm, idx_v, drop_v,
               rows_v, row0_v, *sems):
    gsem, osem = sems[:_NCH], sems[_NCH]
    wid = lax.axis_index("s") * _NC + lax.axis_index("c")
    base = wid * _BPW

    lab_copy = pltpu.async_copy(
        labels_hbm.at[pl.ds(wid * _NCH, _NCH)], idx_v, gsem[0])
    drop_copy = pltpu.async_copy(
        drops_hbm.at[pl.ds(wid * _NCH, _NCH)], drop_v, gsem[1])
    row0_copy = pltpu.async_copy(table_hbm.at[pl.ds(0, 1)], row0_v, osem)
    lab_copy.wait()
    drop_copy.wait()

    lane = lax.iota(jnp.int32, _L)
    gathers = []
    for j in range(_NCH):
        for g in range(_GPC):
            sl = pl.ds(g * _L, _L)
            pos = lane + j * _K + g * _L
            idx_v[j, sl] = jnp.where(drop_v[j, sl] == 1, pos + base,
                                     idx_v[j, sl])
        gathers.append(pltpu.async_copy(
            table_hbm.at[idx_v.at[j]], rows_v.at[pl.ds(j * _K, _K)],
            gsem[j]))

    row0_copy.wait()
    r0 = [row0_v[0, pl.ds(c * _L, _L)] for c in range(_NCOL)]

    out_copies = []
    for j in range(_NCH):
        gathers[j].wait()

        def _fill(g, carry, j=j):
            drop16 = drop_v[j, pl.ds(g * _L, _L)]
            pos = lane + j * _K + g * _L
            tgt = jnp.where(drop16 == 1, pos, _BPW)
            for k in range(_L):
                r = tgt[k]
                for c in range(_NCOL):
                    rows_v[r, pl.ds(c * _L, _L)] = carry[c]
            return carry

        lax.fori_loop(0, _GPC, _fill, tuple(r0))
        out_copies.append(pltpu.async_copy(
            rows_v.at[pl.ds(j * _K, _K)],
            out_hbm.at[pl.ds(base + j * _K, _K)], osem))
    for c in out_copies:
        c.wait()


@jax.jit
def _embed(labels, force_drop_ids, embedding_table):
    mesh = plsc.VectorSubcoreMesh(core_axis_name="c", subcore_axis_name="s")
    return pl.kernel(
        _sc_kernel,
        mesh=mesh,
        out_type=jax.ShapeDtypeStruct((BATCH, HIDDEN_DIM), jnp.float32),
        scratch_types=[
            pltpu.VMEM((_NCH, _K), jnp.int32),
            pltpu.VMEM((_NCH, _K), jnp.int32),
            pltpu.VMEM((_BPW + 1, HIDDEN_DIM), jnp.float32),
            pltpu.VMEM((1, HIDDEN_DIM), jnp.float32),
        ] + [pltpu.SemaphoreType.DMA] * (_NCH + 1),
    )(labels.reshape(BATCH // _K, _K), force_drop_ids.reshape(BATCH // _K, _K),
      embedding_table)


def kernel(labels, train, force_drop_ids, embedding_table):
    del train
    return _embed(labels.astype(jnp.int32), force_drop_ids.astype(jnp.int32),
                  embedding_table)
